# baseline (speedup 1.0000x reference)
import functools

import jax
import jax.numpy as jnp
from jax import lax
from jax.experimental import pallas as pl
from jax.experimental.pallas import tpu as pltpu
from jax.experimental.pallas import tpu_sc as plsc

VOCAB = 100000
DIM = 64
BATCH = 4096
CTX = 20

NC = 2
NS = 16
NW = NC * NS
BPW = BATCH // NW
HALF = CTX // 2
LANES = 16


def _sc_mean_body(idx_hbm, emb_hbm, out_hbm, idx_v, rows_v, acc_v, sem):
    wid = lax.axis_index("s") * NC + lax.axis_index("c")
    pltpu.sync_copy(idx_hbm.at[wid], idx_v)

    for half in range(CTX // HALF):
        copies = [
            pltpu.make_async_copy(
                emb_hbm.at[idx_v.at[half * HALF + j]], rows_v.at[j], sem)
            for j in range(HALF)
        ]
        for c in copies:
            c.start()
        for c in copies:
            c.wait()

        def body(r, carry):
            for k in range(DIM // LANES):
                s = pl.ds(k * LANES, LANES)
                acc = rows_v[0, r, s]
                for j in range(1, HALF):
                    acc = acc + rows_v[j, r, s]
                if half == 0:
                    acc_v[r, s] = acc
                else:
                    acc_v[r, s] = (acc_v[r, s] + acc) * (1.0 / CTX)
            return carry

        lax.fori_loop(0, BPW, body, 0, unroll=False)

    pltpu.sync_copy(acc_v, out_hbm.at[pl.ds(wid * BPW, BPW)])


@functools.cache
def _sc_mean():
    return pl.kernel(
        _sc_mean_body,
        out_type=jax.ShapeDtypeStruct((BATCH, DIM), jnp.float32),
        mesh=plsc.VectorSubcoreMesh(
            core_axis_name="c", subcore_axis_name="s",
            num_cores=NC, num_subcores=NS),
        scratch_types=[
            pltpu.VMEM((CTX, BPW), jnp.int32),
            pltpu.VMEM((HALF, BPW, DIM), jnp.float32),
            pltpu.VMEM((BPW, DIM), jnp.float32),
            pltpu.SemaphoreType.DMA,
        ],
        compiler_params=pltpu.CompilerParams(use_tc_tiling_on_sc=False),
    )


VT = 1024
NV = (VOCAB + VT - 1) // VT


def _mm_body(x_ref, w_ref, b_ref, o_ref):
    o_ref[...] = (
        jnp.dot(x_ref[...], w_ref[...], preferred_element_type=jnp.float32)
        + b_ref[...]
    )


def _project(mean, W, b2):
    return pl.pallas_call(
        _mm_body,
        grid=(NV,),
        in_specs=[
            pl.BlockSpec((BATCH, DIM), lambda i: (0, 0)),
            pl.BlockSpec((DIM, VT), lambda i: (0, i)),
            pl.BlockSpec((1, VT), lambda i: (0, i)),
        ],
        out_specs=pl.BlockSpec((BATCH, VT), lambda i: (0, i)),
        out_shape=jax.ShapeDtypeStruct((BATCH, VOCAB), jnp.float32),
    )(mean, W, b2)


def kernel(context_indices, emb, W, b):
    idx = context_indices.astype(jnp.int32).reshape(NW, BPW, CTX).swapaxes(1, 2)
    mean = _sc_mean()(idx, emb)
    return _project(mean, W, b.reshape(1, VOCAB))


# speedup vs baseline: 1.0017x; 1.0017x over previous
import functools

import jax
import jax.numpy as jnp
from jax import lax
from jax.experimental import pallas as pl
from jax.experimental.pallas import tpu as pltpu
from jax.experimental.pallas import tpu_sc as plsc

VOCAB = 100000
DIM = 64
BATCH = 4096
CTX = 20

NC = 2
NS = 16
NW = NC * NS
BPW = BATCH // NW
HALF = CTX // 2
LANES = 16


def _sc_mean_body(idx_hbm, emb_hbm, out_hbm, idx_v, rows_v, acc_v, sem):
    wid = lax.axis_index("s") * NC + lax.axis_index("c")
    pltpu.sync_copy(idx_hbm.at[wid], idx_v)

    for half in range(CTX // HALF):
        copies = [
            pltpu.make_async_copy(
                emb_hbm.at[idx_v.at[half * HALF + j]], rows_v.at[j], sem)
            for j in range(HALF)
        ]
        for c in copies:
            c.start()
        for c in copies:
            c.wait()

        def body(r, carry):
            for k in range(DIM // LANES):
                s = pl.ds(k * LANES, LANES)
                acc = rows_v[0, r, s]
                for j in range(1, HALF):
                    acc = acc + rows_v[j, r, s]
                if half == 0:
                    acc_v[r, s] = acc
                else:
                    acc_v[r, s] = (acc_v[r, s] + acc) * (1.0 / CTX)
            return carry

        lax.fori_loop(0, BPW, body, 0, unroll=False)

    pltpu.sync_copy(acc_v, out_hbm.at[pl.ds(wid * BPW, BPW)])


@functools.cache
def _sc_mean():
    return pl.kernel(
        _sc_mean_body,
        out_type=jax.ShapeDtypeStruct((BATCH, DIM), jnp.float32),
        mesh=plsc.VectorSubcoreMesh(
            core_axis_name="c", subcore_axis_name="s",
            num_cores=NC, num_subcores=NS),
        scratch_types=[
            pltpu.VMEM((CTX, BPW), jnp.int32),
            pltpu.VMEM((HALF, BPW, DIM), jnp.float32),
            pltpu.VMEM((BPW, DIM), jnp.float32),
            pltpu.SemaphoreType.DMA,
        ],
        compiler_params=pltpu.CompilerParams(use_tc_tiling_on_sc=False),
    )


VT = 1024
NV = (VOCAB + VT - 1) // VT


def _mm_body(x_ref, w_ref, b_ref, o_ref):
    o_ref[...] = (
        jnp.dot(x_ref[...].astype(jnp.bfloat16), w_ref[...].astype(jnp.bfloat16),
                preferred_element_type=jnp.float32)
        + b_ref[...]
    )


def _project(mean, W, b2):
    return pl.pallas_call(
        _mm_body,
        grid=(NV,),
        in_specs=[
            pl.BlockSpec((BATCH, DIM), lambda i: (0, 0)),
            pl.BlockSpec((DIM, VT), lambda i: (0, i)),
            pl.BlockSpec((1, VT), lambda i: (0, i)),
        ],
        out_specs=pl.BlockSpec((BATCH, VT), lambda i: (0, i)),
        out_shape=jax.ShapeDtypeStruct((BATCH, VOCAB), jnp.float32),
    )(mean, W, b2)


def kernel(context_indices, emb, W, b):
    idx = context_indices.astype(jnp.int32).reshape(NW, BPW, CTX).swapaxes(1, 2)
    mean = _sc_mean()(idx, emb)
    return _project(mean, W, b.reshape(1, VOCAB))
